# Initial kernel scaffold; baseline (speedup 1.0000x reference)
import jax
import jax.numpy as jnp
from jax.experimental import pallas as pl


def kernel(x, label, edge_index, is_direct):
    raise NotImplementedError("write your pallas kernel here")


# speedup vs baseline: 1.5454x; 1.5454x over previous
import functools

import jax
import jax.numpy as jnp
from jax.experimental import pallas as pl


def _segsum(src, index, n):
    return jax.ops.segment_sum(src, index, num_segments=n)


def _norm_concat_body(x2, x1, h3, h2, h1, h4, o):
    parts = (x2, x1, h3, h2, h1, h4)
    for i, p in enumerate(parts):
        v = p[...]
        s = jnp.sum(v, axis=1, keepdims=True) + 1e-5
        o[:, i * 16:(i + 1) * 16] = v / s


def _norm_concat(x2, x1, h3, h2, h1, h4):
    n = x2.shape[0]
    rows = 1000
    grid = (n // rows,)
    in_spec = pl.BlockSpec((rows, 16), lambda i: (i, 0))
    out_spec = pl.BlockSpec((rows, 96), lambda i: (i, 0))
    return pl.pallas_call(
        _norm_concat_body,
        grid=grid,
        in_specs=[in_spec] * 6,
        out_specs=out_spec,
        out_shape=jax.ShapeDtypeStruct((n, 96), jnp.float32),
    )(x2, x1, h3, h2, h1, h4)


def kernel(x, label, edge_index, is_direct):
    n = label.shape[0]
    row = edge_index[0]
    col = edge_index[1]
    mask = (row != col).astype(jnp.float32)

    hist_row = _segsum(mask, row, n)
    hist_col = _segsum(mask, col, n)
    out_deg = hist_row + 1e-9
    in_deg = hist_col + 1e-9
    dis = jax.lax.rsqrt(in_deg)
    dos = jax.lax.rsqrt(out_deg)
    norm = dis[col] * dos[row] * mask

    def conv(src_vals, s_idx, d_idx, w):
        return _segsum(w[:, None] * src_vals[s_idx], d_idx, n)

    xl = label
    h1 = conv(xl, col, row, norm)
    h2 = conv(xl, row, col, norm)
    h3 = conv(h2, col, row, norm)
    norm2 = norm * norm
    re_a = _segsum(norm2, row, n)
    h3 = h3 - label * re_a[:, None]
    h4 = conv(h1, row, col, norm)
    re_b = _segsum(norm2, col, n)
    h4 = h4 - label * re_b[:, None]

    both_r = jnp.concatenate([row, col])
    both_c = jnp.concatenate([col, row])
    order = jnp.lexsort((both_c, both_r))
    rs = both_r[order]
    cs = both_c[order]
    first = jnp.concatenate(
        [jnp.ones((1,), dtype=bool), (rs[1:] != rs[:-1]) | (cs[1:] != cs[:-1])]
    )
    uw = jnp.where(first & (rs != cs), 1.0, 0.0)
    deg_u = _segsum(uw, rs, n) + 1e-9
    du = jax.lax.rsqrt(deg_u)
    normu = du[cs] * du[rs] * uw
    x1 = conv(xl, rs, cs, normu)
    x2 = conv(x1, rs, cs, normu)
    re3 = _segsum(normu * normu, cs, n)
    x2 = x2 - label * re3[:, None]

    return _norm_concat(x2, x1, h3, h2, h1, h4)


# speedup vs baseline: 28.8595x; 18.6747x over previous
import functools

import jax
import jax.numpy as jnp
from jax import lax
from jax.experimental import pallas as pl
from jax.experimental.pallas import tpu as pltpu, tpu_sc as plsc

_N = 100000
_E = 3200000
_CH = 80
_K = 8
_BLK = _CH * _K
_NSC = 16
_RPT = 6272
_NA = _RPT * _NSC
_TRASH = _N
_WB = 392
_EC_CONV = 3276800
_EC_HIST3 = 1638400

_mesh = plsc.VectorSubcoreMesh(core_axis_name="c", subcore_axis_name="s")
_params = pltpu.CompilerParams(use_tc_tiling_on_sc=False)


def _stage_idx(idx_h, buf, sem, sid, nblk, blk):
    off = sid * (nblk * _K) + blk * _K
    return pltpu.async_copy(idx_h.at[pl.ds(off, _K), :], buf, sem)


def _zero_fill2(zb):
    def st(i, c):
        zb[i, :] = jnp.zeros((16,), jnp.float32)
        return c
    lax.fori_loop(0, zb.shape[0], st, 0)


def _zero_fill1(zb):
    def st(i, c):
        zb[pl.ds(i * 16, 16)] = jnp.zeros((16,), jnp.float32)
        return c
    lax.fori_loop(0, zb.shape[0] // 16, st, 0)


def _make_conv(nblk, with_scalar):
    out_type = [jax.ShapeDtypeStruct((_NA, 16), jnp.float32)] * 2
    scratch = [
        pltpu.VMEM_SHARED((_NA, 16), jnp.float32),
        pltpu.VMEM((_K, _CH), jnp.int32),
        pltpu.VMEM((_K, _CH), jnp.int32),
        pltpu.VMEM((_K, _CH, 16), jnp.float32),
        pltpu.VMEM((_WB, 16), jnp.float32),
        pltpu.SemaphoreType.DMA,
        pltpu.SemaphoreType.DMA,
        pltpu.SemaphoreType.DMA,
    ]
    if with_scalar:
        out_type += [jax.ShapeDtypeStruct((_NA,), jnp.float32)] * 2
        scratch += [
            pltpu.VMEM_SHARED((_NA,), jnp.float32),
            pltpu.VMEM((_K, _CH), jnp.float32),
            pltpu.VMEM((_WB,), jnp.float32),
        ]

    @functools.partial(pl.kernel, mesh=_mesh, out_type=out_type,
                       scratch_types=scratch, compiler_params=_params)
    def conv_kernel(*refs):
        if with_scalar:
            (t16a, t16b, t1a, t1b, sA, dA, sB, dB,
             oA, oB, o1A, o1B,
             acc16, sidxv, didxv, rows, zb16, sem_i, sem_g, sem_s,
             acc1, scal, zb1) = refs
        else:
            (t16a, t16b, sA, dA, sB, dB,
             oA, oB,
             acc16, sidxv, didxv, rows, zb16, sem_i, sem_g, sem_s) = refs
            t1a = t1b = o1A = o1B = acc1 = scal = zb1 = None
        cid = lax.axis_index("c")
        sid = lax.axis_index("s")
        r0 = sid * _RPT

        _zero_fill2(zb16)
        for i in range(_RPT // _WB):
            pltpu.sync_copy(zb16, acc16.at[pl.ds(r0 + i * _WB, _WB), :])
        if with_scalar:
            _zero_fill1(zb1)
            for i in range(_RPT // _WB):
                pltpu.sync_copy(zb1, acc1.at[pl.ds(r0 + i * _WB, _WB)])
        plsc.subcore_barrier()

        def job(t16, t1, sidx_h, didx_h):
            def blk_body(blk, carry):
                c1 = _stage_idx(sidx_h, sidxv, sem_i, sid, nblk, blk)
                c2 = _stage_idx(didx_h, didxv, sem_i, sid, nblk, blk)
                c1.wait()
                c2.wait()
                gh = []
                for j in range(_K):
                    gh.append(pltpu.async_copy(
                        t16.at[sidxv.at[j]], rows.at[j], sem_g))
                    if with_scalar:
                        gh.append(pltpu.async_copy(
                            t1.at[sidxv.at[j]], scal.at[j], sem_g))
                for h in gh:
                    h.wait()
                sh = []
                for j in range(_K):
                    sh.append(pltpu.async_copy(
                        rows.at[j], acc16.at[didxv.at[j]], sem_s, add=True))
                    if with_scalar:
                        sh.append(pltpu.async_copy(
                            scal.at[j], acc1.at[didxv.at[j]], sem_s, add=True))
                for h in sh:
                    h.wait()
                return carry
            lax.fori_loop(0, nblk, blk_body, 0)

        @pl.when(cid == 0)
        def _():
            job(t16a, t1a, sA, dA)

        @pl.when(cid == 1)
        def _():
            job(t16b, t1b, sB, dB)

        plsc.subcore_barrier()

        def writeback(o16, o1):
            for i in range(_RPT // _WB):
                pltpu.sync_copy(acc16.at[pl.ds(r0 + i * _WB, _WB), :], zb16)
                pltpu.sync_copy(zb16, o16.at[pl.ds(r0 + i * _WB, _WB), :])
            if with_scalar:
                for i in range(_RPT // _WB):
                    pltpu.sync_copy(acc1.at[pl.ds(r0 + i * _WB, _WB)], zb1)
                    pltpu.sync_copy(zb1, o1.at[pl.ds(r0 + i * _WB, _WB)])

        @pl.when(cid == 0)
        def _():
            writeback(oA, o1A)

        @pl.when(cid == 1)
        def _():
            writeback(oB, o1B)

    return conv_kernel


def _make_hist3(nblk):
    out_type = [jax.ShapeDtypeStruct((_NA,), jnp.float32)] * 6
    scratch = [
        pltpu.VMEM_SHARED((_NA,), jnp.float32),
        pltpu.VMEM_SHARED((_NA,), jnp.float32),
        pltpu.VMEM_SHARED((_NA,), jnp.float32),
        pltpu.VMEM((_K, _CH), jnp.int32),
        pltpu.VMEM((_K, _CH), jnp.int32),
        pltpu.VMEM((_K, _CH), jnp.float32),
        pltpu.VMEM((_K, _CH), jnp.float32),
        pltpu.VMEM((_WB,), jnp.float32),
        pltpu.SemaphoreType.DMA,
        pltpu.SemaphoreType.DMA,
    ]

    @functools.partial(pl.kernel, mesh=_mesh, out_type=out_type,
                       scratch_types=scratch, compiler_params=_params)
    def hist_kernel(sA, dA, sB, dB,
                    odA, odB, idA, idB, slA, slB,
                    accA, accB, accC, sidxv, didxv, maskv, selfv, zb1,
                    sem_i, sem_s):
        cid = lax.axis_index("c")
        sid = lax.axis_index("s")
        r0 = sid * _RPT
        _zero_fill1(zb1)
        for i in range(_RPT // _WB):
            pltpu.sync_copy(zb1, accA.at[pl.ds(r0 + i * _WB, _WB)])
            pltpu.sync_copy(zb1, accB.at[pl.ds(r0 + i * _WB, _WB)])
            pltpu.sync_copy(zb1, accC.at[pl.ds(r0 + i * _WB, _WB)])
        plsc.subcore_barrier()

        def job(sidx_h, didx_h):
            def blk_body(blk, carry):
                c1 = _stage_idx(sidx_h, sidxv, sem_i, sid, nblk, blk)
                c2 = _stage_idx(didx_h, didxv, sem_i, sid, nblk, blk)
                c1.wait()
                c2.wait()
                for j in range(_K):
                    for v in range(_CH // 16):
                        r = sidxv[j, pl.ds(v * 16, 16)]
                        c = didxv[j, pl.ds(v * 16, 16)]
                        m = jnp.where(r == c, 0.0, 1.0).astype(jnp.float32)
                        maskv[j, pl.ds(v * 16, 16)] = m
                        selfv[j, pl.ds(v * 16, 16)] = 1.0 - m
                sh = []
                for j in range(_K):
                    sh.append(pltpu.async_copy(
                        maskv.at[j], accA.at[sidxv.at[j]], sem_s, add=True))
                    sh.append(pltpu.async_copy(
                        maskv.at[j], accB.at[didxv.at[j]], sem_s, add=True))
                    sh.append(pltpu.async_copy(
                        selfv.at[j], accC.at[sidxv.at[j]], sem_s, add=True))
                for h in sh:
                    h.wait()
                return carry
            lax.fori_loop(0, nblk, blk_body, 0)

        @pl.when(cid == 0)
        def _():
            job(sA, dA)

        @pl.when(cid == 1)
        def _():
            job(sB, dB)

        plsc.subcore_barrier()

        def writeback(od, idg, sl):
            for i in range(_RPT // _WB):
                pltpu.sync_copy(accA.at[pl.ds(r0 + i * _WB, _WB)], zb1)
                pltpu.sync_copy(zb1, od.at[pl.ds(r0 + i * _WB, _WB)])
                pltpu.sync_copy(accB.at[pl.ds(r0 + i * _WB, _WB)], zb1)
                pltpu.sync_copy(zb1, idg.at[pl.ds(r0 + i * _WB, _WB)])
                pltpu.sync_copy(accC.at[pl.ds(r0 + i * _WB, _WB)], zb1)
                pltpu.sync_copy(zb1, sl.at[pl.ds(r0 + i * _WB, _WB)])

        @pl.when(cid == 0)
        def _():
            writeback(odA, idA, slA)

        @pl.when(cid == 1)
        def _():
            writeback(odB, idB, slB)

    return hist_kernel


def _make_hist1(nblk):
    out_type = [jax.ShapeDtypeStruct((_NA,), jnp.float32)] * 2
    scratch = [
        pltpu.VMEM_SHARED((_NA,), jnp.float32),
        pltpu.VMEM((_K, _CH), jnp.int32),
        pltpu.VMEM((_CH,), jnp.float32),
        pltpu.VMEM((_WB,), jnp.float32),
        pltpu.SemaphoreType.DMA,
        pltpu.SemaphoreType.DMA,
    ]

    @functools.partial(pl.kernel, mesh=_mesh, out_type=out_type,
                       scratch_types=scratch, compiler_params=_params)
    def hist1_kernel(iA, iB, oA, oB, acc, idxv, onesv, zb1, sem_i, sem_s):
        cid = lax.axis_index("c")
        sid = lax.axis_index("s")
        r0 = sid * _RPT
        _zero_fill1(zb1)
        for i in range(_RPT // _WB):
            pltpu.sync_copy(zb1, acc.at[pl.ds(r0 + i * _WB, _WB)])
        for v in range(_CH // 16):
            onesv[pl.ds(v * 16, 16)] = jnp.full((16,), 1.0, jnp.float32)
        plsc.subcore_barrier()

        def job(idx_h):
            def blk_body(blk, carry):
                _stage_idx(idx_h, idxv, sem_i, sid, nblk, blk).wait()
                sh = []
                for j in range(_K):
                    sh.append(pltpu.async_copy(
                        onesv, acc.at[idxv.at[j]], sem_s, add=True))
                for h in sh:
                    h.wait()
                return carry
            lax.fori_loop(0, nblk, blk_body, 0)

        @pl.when(cid == 0)
        def _():
            job(iA)

        @pl.when(cid == 1)
        def _():
            job(iB)

        plsc.subcore_barrier()

        def writeback(o):
            for i in range(_RPT // _WB):
                pltpu.sync_copy(acc.at[pl.ds(r0 + i * _WB, _WB)], zb1)
                pltpu.sync_copy(zb1, o.at[pl.ds(r0 + i * _WB, _WB)])

        @pl.when(cid == 0)
        def _():
            writeback(oA)

        @pl.when(cid == 1)
        def _():
            writeback(oB)

    return hist1_kernel


_conv_plain = _make_conv(_EC_CONV // (_NSC * _BLK), False)
_conv_scal = _make_conv(_EC_CONV // (_NSC * _BLK), True)
_hist3 = _make_hist3(_EC_HIST3 // (_NSC * _BLK))
_hist1 = _make_hist1(_EC_CONV // (_NSC * _BLK))


def _norm_concat_body(x2, x1, h3, h2, h1, h4, o):
    parts = (x2, x1, h3, h2, h1, h4)
    for i, p in enumerate(parts):
        v = p[...]
        s = jnp.sum(v, axis=1, keepdims=True) + 1e-5
        o[:, i * 16:(i + 1) * 16] = v / s


def _norm_concat(x2, x1, h3, h2, h1, h4):
    n = x2.shape[0]
    rows = 1000
    grid = (n // rows,)
    in_spec = pl.BlockSpec((rows, 16), lambda i: (i, 0))
    out_spec = pl.BlockSpec((rows, 96), lambda i: (i, 0))
    return pl.pallas_call(
        _norm_concat_body,
        grid=grid,
        in_specs=[in_spec] * 6,
        out_specs=out_spec,
        out_shape=jax.ShapeDtypeStruct((n, 96), jnp.float32),
    )(x2, x1, h3, h2, h1, h4)


def _chunks(a, ec_pad, fill=0):
    pad = ec_pad - a.shape[0]
    if pad:
        a = jnp.concatenate([a, jnp.full((pad,), fill, jnp.int32)])
    return a.reshape(-1, _CH)


def kernel(x, label, edge_index, is_direct):
    n = label.shape[0]
    e = edge_index.shape[1]
    row = edge_index[0].astype(jnp.int32)
    col = edge_index[1].astype(jnp.int32)

    h = e // 2
    od0, od1, id0, id1, sl0, sl1 = _hist3(
        _chunks(row[:h], _EC_HIST3, _TRASH), _chunks(col[:h], _EC_HIST3, _TRASH),
        _chunks(row[h:], _EC_HIST3, _TRASH), _chunks(col[h:], _EC_HIST3, _TRASH))
    out_deg = (od0 + od1)[:n] + 1e-9
    in_deg = (id0 + id1)[:n] + 1e-9
    s = (sl0 + sl1)[:n]
    dis = lax.rsqrt(in_deg)
    dos = lax.rsqrt(out_deg)

    t1 = dis[:, None] * label
    t2 = dos[:, None] * label
    h1_raw, h2_raw = _conv_plain(
        t1, t2,
        _chunks(col, _EC_CONV), _chunks(row, _EC_CONV, _TRASH),
        _chunks(row, _EC_CONV), _chunks(col, _EC_CONV, _TRASH))
    h1 = dos[:, None] * (h1_raw[:n] - s[:, None] * t1)
    h2 = dis[:, None] * (h2_raw[:n] - s[:, None] * t2)

    t3 = dis[:, None] * h2
    t4 = dos[:, None] * h1
    dis2 = dis * dis
    dos2 = dos * dos
    h3_raw, h4_raw, rea_raw, reb_raw = _conv_scal(
        t3, t4, dis2, dos2,
        _chunks(col, _EC_CONV), _chunks(row, _EC_CONV, _TRASH),
        _chunks(row, _EC_CONV), _chunks(col, _EC_CONV, _TRASH))
    re_a = dos2 * (rea_raw[:n] - s * dis2)
    re_b = dis2 * (reb_raw[:n] - s * dos2)
    h3 = dos[:, None] * (h3_raw[:n] - s[:, None] * t3) - label * re_a[:, None]
    h4 = dis[:, None] * (h4_raw[:n] - s[:, None] * t4) - label * re_b[:, None]

    br = jnp.concatenate([row, col])
    bc = jnp.concatenate([col, row])
    order = jnp.lexsort((bc, br))
    rs = br[order]
    cs = bc[order]
    first = jnp.concatenate(
        [jnp.ones((1,), bool), (rs[1:] != rs[:-1]) | (cs[1:] != cs[:-1])])
    keep = first & (rs != cs)
    trash = jnp.int32(_TRASH)
    rs_k = jnp.where(keep, rs, trash)
    cs_k = jnp.where(keep, cs, trash)

    hh = e
    d0, d1 = _hist1(_chunks(rs_k[:hh], _EC_CONV, _TRASH),
                    _chunks(rs_k[hh:], _EC_CONV, _TRASH))
    deg_u = (d0 + d1)[:n] + 1e-9
    du = lax.rsqrt(deg_u)
    du2 = du * du

    t5 = du[:, None] * label
    p0, p1 = _conv_plain(
        t5, t5,
        _chunks(rs[:hh], _EC_CONV), _chunks(cs_k[:hh], _EC_CONV, _TRASH),
        _chunks(rs[hh:], _EC_CONV), _chunks(cs_k[hh:], _EC_CONV, _TRASH))
    x1 = du[:, None] * (p0 + p1)[:n]

    t6 = du[:, None] * x1
    q0, q1, r3a, r3b = _conv_scal(
        t6, t6, du2, du2,
        _chunks(rs[:hh], _EC_CONV), _chunks(cs_k[:hh], _EC_CONV, _TRASH),
        _chunks(rs[hh:], _EC_CONV), _chunks(cs_k[hh:], _EC_CONV, _TRASH))
    re3 = du2 * (r3a + r3b)[:n]
    x2 = du[:, None] * (q0 + q1)[:n] - label * re3[:, None]

    return _norm_concat(x2, x1, h3, h2, h1, h4)


# speedup vs baseline: 30.1161x; 1.0435x over previous
import functools

import jax
import jax.numpy as jnp
from jax import lax
from jax.experimental import pallas as pl
from jax.experimental.pallas import tpu as pltpu, tpu_sc as plsc

_N = 100000
_E = 3200000
_CH = 80
_K = 8
_BLK = _CH * _K
_NSC = 16
_RPT = 6272
_NA = _RPT * _NSC
_TRASH = _N
_WB = 392
_EC_CONV = 3276800
_EC_HIST3 = 1638400

_mesh = plsc.VectorSubcoreMesh(core_axis_name="c", subcore_axis_name="s")
_params = pltpu.CompilerParams(use_tc_tiling_on_sc=False)


def _stage_idx(idx_h, buf, sem, sid, nblk, blk):
    off = sid * (nblk * _K) + blk * _K
    return pltpu.async_copy(idx_h.at[pl.ds(off, _K), :], buf, sem)


def _zero_fill2(zb):
    def st(i, c):
        zb[i, :] = jnp.zeros((16,), jnp.float32)
        return c
    lax.fori_loop(0, zb.shape[0], st, 0)


def _zero_fill1(zb):
    def st(i, c):
        zb[pl.ds(i * 16, 16)] = jnp.zeros((16,), jnp.float32)
        return c
    lax.fori_loop(0, zb.shape[0] // 16, st, 0)


def _make_conv(nblk, with_scalar, ch, wb):
    out_type = [jax.ShapeDtypeStruct((_NA, 16), jnp.float32)] * 2
    scratch = [
        pltpu.VMEM_SHARED((_NA, 16), jnp.float32),
        pltpu.VMEM((2, _K, ch), jnp.int32),
        pltpu.VMEM((2, _K, ch), jnp.int32),
        pltpu.VMEM((2, _K, ch, 16), jnp.float32),
        pltpu.VMEM((wb, 16), jnp.float32),
        pltpu.SemaphoreType.DMA,
        pltpu.SemaphoreType.DMA,
        pltpu.SemaphoreType.DMA,
        pltpu.SemaphoreType.DMA,
        pltpu.SemaphoreType.DMA,
        pltpu.SemaphoreType.DMA,
    ]
    if with_scalar:
        out_type += [jax.ShapeDtypeStruct((_NA,), jnp.float32)] * 2
        scratch += [
            pltpu.VMEM_SHARED((_NA,), jnp.float32),
            pltpu.VMEM((2, _K, ch), jnp.float32),
            pltpu.VMEM((wb,), jnp.float32),
        ]
    nhalf = nblk // 2

    @functools.partial(pl.kernel, mesh=_mesh, out_type=out_type,
                       scratch_types=scratch, compiler_params=_params)
    def conv_kernel(*refs):
        if with_scalar:
            (t16a, t16b, t1a, t1b, sA, dA, sB, dB,
             oA, oB, o1A, o1B,
             acc16, sidxv, didxv, rows, zb16,
             sem_i0, sem_i1, sem_g0, sem_g1, sem_s0, sem_s1,
             acc1, scal, zb1) = refs
        else:
            (t16a, t16b, sA, dA, sB, dB,
             oA, oB,
             acc16, sidxv, didxv, rows, zb16,
             sem_i0, sem_i1, sem_g0, sem_g1, sem_s0, sem_s1) = refs
            t1a = t1b = o1A = o1B = acc1 = scal = zb1 = None
        sem_i = (sem_i0, sem_i1)
        sem_g = (sem_g0, sem_g1)
        sem_s = (sem_s0, sem_s1)
        cid = lax.axis_index("c")
        sid = lax.axis_index("s")
        r0 = sid * _RPT

        _zero_fill2(zb16)
        for i in range(_RPT // wb):
            pltpu.sync_copy(zb16, acc16.at[pl.ds(r0 + i * wb, wb), :])
        if with_scalar:
            _zero_fill1(zb1)
            for i in range(_RPT // wb):
                pltpu.sync_copy(zb1, acc1.at[pl.ds(r0 + i * wb, wb)])
        plsc.subcore_barrier()

        def job(t16, t1, sidx_h, didx_h):
            def stage(p, blk):
                off = sid * (nblk * _K) + blk * _K
                h1 = pltpu.async_copy(
                    sidx_h.at[pl.ds(off, _K), :], sidxv.at[p], sem_i[p])
                h2 = pltpu.async_copy(
                    didx_h.at[pl.ds(off, _K), :], didxv.at[p], sem_i[p])
                return h1, h2

            def gathers(p):
                ds = []
                for j in range(_K):
                    ds.append(pltpu.make_async_copy(
                        t16.at[sidxv.at[p, j]], rows.at[p, j], sem_g[p]))
                    if with_scalar:
                        ds.append(pltpu.make_async_copy(
                            t1.at[sidxv.at[p, j]], scal.at[p, j], sem_g[p]))
                return ds

            def scatters(p):
                hs = []
                for j in range(_K):
                    hs.append(pltpu.async_copy(
                        rows.at[p, j], acc16.at[didxv.at[p, j]],
                        sem_s[p], add=True))
                    if with_scalar:
                        hs.append(pltpu.async_copy(
                            scal.at[p, j], acc1.at[didxv.at[p, j]],
                            sem_s[p], add=True))
                return hs

            h1, h2 = stage(0, 0)
            h1.wait()
            h2.wait()
            for d in gathers(0):
                d.start()

            def pair_body(i, carry):
                blk1 = 2 * i + 1
                s1a, s1b = stage(1, blk1)
                for d in gathers(0):
                    d.wait()
                sc0 = scatters(0)
                s1a.wait()
                s1b.wait()
                for d in gathers(1):
                    d.start()
                for h in sc0:
                    h.wait()
                for d in gathers(1):
                    d.wait()
                sc1 = scatters(1)

                @pl.when(i + 1 < nhalf)
                def _():
                    n1, n2 = stage(0, 2 * i + 2)
                    n1.wait()
                    n2.wait()
                    for d in gathers(0):
                        d.start()

                for h in sc1:
                    h.wait()
                return carry

            lax.fori_loop(0, nhalf, pair_body, 0)

        @pl.when(cid == 0)
        def _():
            job(t16a, t1a, sA, dA)

        @pl.when(cid == 1)
        def _():
            job(t16b, t1b, sB, dB)

        plsc.subcore_barrier()

        def writeback(o16, o1):
            for i in range(_RPT // wb):
                pltpu.sync_copy(acc16.at[pl.ds(r0 + i * wb, wb), :], zb16)
                pltpu.sync_copy(zb16, o16.at[pl.ds(r0 + i * wb, wb), :])
            if with_scalar:
                for i in range(_RPT // wb):
                    pltpu.sync_copy(acc1.at[pl.ds(r0 + i * wb, wb)], zb1)
                    pltpu.sync_copy(zb1, o1.at[pl.ds(r0 + i * wb, wb)])

        @pl.when(cid == 0)
        def _():
            writeback(oA, o1A)

        @pl.when(cid == 1)
        def _():
            writeback(oB, o1B)

    return conv_kernel


def _make_hist3(nblk):
    out_type = [jax.ShapeDtypeStruct((_NA,), jnp.float32)] * 6
    scratch = [
        pltpu.VMEM_SHARED((_NA,), jnp.float32),
        pltpu.VMEM_SHARED((_NA,), jnp.float32),
        pltpu.VMEM_SHARED((_NA,), jnp.float32),
        pltpu.VMEM((_K, _CH), jnp.int32),
        pltpu.VMEM((_K, _CH), jnp.int32),
        pltpu.VMEM((_K, _CH), jnp.float32),
        pltpu.VMEM((_K, _CH), jnp.float32),
        pltpu.VMEM((_WB,), jnp.float32),
        pltpu.SemaphoreType.DMA,
        pltpu.SemaphoreType.DMA,
    ]

    @functools.partial(pl.kernel, mesh=_mesh, out_type=out_type,
                       scratch_types=scratch, compiler_params=_params)
    def hist_kernel(sA, dA, sB, dB,
                    odA, odB, idA, idB, slA, slB,
                    accA, accB, accC, sidxv, didxv, maskv, selfv, zb1,
                    sem_i, sem_s):
        cid = lax.axis_index("c")
        sid = lax.axis_index("s")
        r0 = sid * _RPT
        _zero_fill1(zb1)
        for i in range(_RPT // _WB):
            pltpu.sync_copy(zb1, accA.at[pl.ds(r0 + i * _WB, _WB)])
            pltpu.sync_copy(zb1, accB.at[pl.ds(r0 + i * _WB, _WB)])
            pltpu.sync_copy(zb1, accC.at[pl.ds(r0 + i * _WB, _WB)])
        plsc.subcore_barrier()

        def job(sidx_h, didx_h):
            def blk_body(blk, carry):
                c1 = _stage_idx(sidx_h, sidxv, sem_i, sid, nblk, blk)
                c2 = _stage_idx(didx_h, didxv, sem_i, sid, nblk, blk)
                c1.wait()
                c2.wait()
                for j in range(_K):
                    for v in range(_CH // 16):
                        r = sidxv[j, pl.ds(v * 16, 16)]
                        c = didxv[j, pl.ds(v * 16, 16)]
                        m = jnp.where(r == c, 0.0, 1.0).astype(jnp.float32)
                        maskv[j, pl.ds(v * 16, 16)] = m
                        selfv[j, pl.ds(v * 16, 16)] = 1.0 - m
                sh = []
                for j in range(_K):
                    sh.append(pltpu.async_copy(
                        maskv.at[j], accA.at[sidxv.at[j]], sem_s, add=True))
                    sh.append(pltpu.async_copy(
                        maskv.at[j], accB.at[didxv.at[j]], sem_s, add=True))
                    sh.append(pltpu.async_copy(
                        selfv.at[j], accC.at[sidxv.at[j]], sem_s, add=True))
                for h in sh:
                    h.wait()
                return carry
            lax.fori_loop(0, nblk, blk_body, 0)

        @pl.when(cid == 0)
        def _():
            job(sA, dA)

        @pl.when(cid == 1)
        def _():
            job(sB, dB)

        plsc.subcore_barrier()

        def writeback(od, idg, sl):
            for i in range(_RPT // _WB):
                pltpu.sync_copy(accA.at[pl.ds(r0 + i * _WB, _WB)], zb1)
                pltpu.sync_copy(zb1, od.at[pl.ds(r0 + i * _WB, _WB)])
                pltpu.sync_copy(accB.at[pl.ds(r0 + i * _WB, _WB)], zb1)
                pltpu.sync_copy(zb1, idg.at[pl.ds(r0 + i * _WB, _WB)])
                pltpu.sync_copy(accC.at[pl.ds(r0 + i * _WB, _WB)], zb1)
                pltpu.sync_copy(zb1, sl.at[pl.ds(r0 + i * _WB, _WB)])

        @pl.when(cid == 0)
        def _():
            writeback(odA, idA, slA)

        @pl.when(cid == 1)
        def _():
            writeback(odB, idB, slB)

    return hist_kernel


def _make_hist1(nblk):
    out_type = [jax.ShapeDtypeStruct((_NA,), jnp.float32)] * 2
    scratch = [
        pltpu.VMEM_SHARED((_NA,), jnp.float32),
        pltpu.VMEM((_K, _CH), jnp.int32),
        pltpu.VMEM((_CH,), jnp.float32),
        pltpu.VMEM((_WB,), jnp.float32),
        pltpu.SemaphoreType.DMA,
        pltpu.SemaphoreType.DMA,
    ]

    @functools.partial(pl.kernel, mesh=_mesh, out_type=out_type,
                       scratch_types=scratch, compiler_params=_params)
    def hist1_kernel(iA, iB, oA, oB, acc, idxv, onesv, zb1, sem_i, sem_s):
        cid = lax.axis_index("c")
        sid = lax.axis_index("s")
        r0 = sid * _RPT
        _zero_fill1(zb1)
        for i in range(_RPT // _WB):
            pltpu.sync_copy(zb1, acc.at[pl.ds(r0 + i * _WB, _WB)])
        for v in range(_CH // 16):
            onesv[pl.ds(v * 16, 16)] = jnp.full((16,), 1.0, jnp.float32)
        plsc.subcore_barrier()

        def job(idx_h):
            def blk_body(blk, carry):
                _stage_idx(idx_h, idxv, sem_i, sid, nblk, blk).wait()
                sh = []
                for j in range(_K):
                    sh.append(pltpu.async_copy(
                        onesv, acc.at[idxv.at[j]], sem_s, add=True))
                for h in sh:
                    h.wait()
                return carry
            lax.fori_loop(0, nblk, blk_body, 0)

        @pl.when(cid == 0)
        def _():
            job(iA)

        @pl.when(cid == 1)
        def _():
            job(iB)

        plsc.subcore_barrier()

        def writeback(o):
            for i in range(_RPT // _WB):
                pltpu.sync_copy(acc.at[pl.ds(r0 + i * _WB, _WB)], zb1)
                pltpu.sync_copy(zb1, o.at[pl.ds(r0 + i * _WB, _WB)])

        @pl.when(cid == 0)
        def _():
            writeback(oA)

        @pl.when(cid == 1)
        def _():
            writeback(oB)

    return hist1_kernel


_CH_S = 64
_WB_S = 224
_conv_plain = _make_conv(_EC_CONV // (_NSC * _K * _CH), False, _CH, _WB)
_conv_scal = _make_conv(_EC_CONV // (_NSC * _K * _CH_S), True, _CH_S, _WB_S)
_hist3 = _make_hist3(_EC_HIST3 // (_NSC * _BLK))
_hist1 = _make_hist1(_EC_CONV // (_NSC * _BLK))


def _norm_concat_body(x2, x1, h3, h2, h1, h4, o):
    parts = (x2, x1, h3, h2, h1, h4)
    for i, p in enumerate(parts):
        v = p[...]
        s = jnp.sum(v, axis=1, keepdims=True) + 1e-5
        o[:, i * 16:(i + 1) * 16] = v / s


def _norm_concat(x2, x1, h3, h2, h1, h4):
    n = x2.shape[0]
    rows = 1000
    grid = (n // rows,)
    in_spec = pl.BlockSpec((rows, 16), lambda i: (i, 0))
    out_spec = pl.BlockSpec((rows, 96), lambda i: (i, 0))
    return pl.pallas_call(
        _norm_concat_body,
        grid=grid,
        in_specs=[in_spec] * 6,
        out_specs=out_spec,
        out_shape=jax.ShapeDtypeStruct((n, 96), jnp.float32),
    )(x2, x1, h3, h2, h1, h4)


def _chunks(a, ec_pad, fill=0, ch=_CH):
    pad = ec_pad - a.shape[0]
    if pad:
        a = jnp.concatenate([a, jnp.full((pad,), fill, jnp.int32)])
    return a.reshape(-1, ch)


def kernel(x, label, edge_index, is_direct):
    n = label.shape[0]
    e = edge_index.shape[1]
    row = edge_index[0].astype(jnp.int32)
    col = edge_index[1].astype(jnp.int32)

    h = e // 2
    od0, od1, id0, id1, sl0, sl1 = _hist3(
        _chunks(row[:h], _EC_HIST3, _TRASH), _chunks(col[:h], _EC_HIST3, _TRASH),
        _chunks(row[h:], _EC_HIST3, _TRASH), _chunks(col[h:], _EC_HIST3, _TRASH))
    out_deg = (od0 + od1)[:n] + 1e-9
    in_deg = (id0 + id1)[:n] + 1e-9
    s = (sl0 + sl1)[:n]
    dis = lax.rsqrt(in_deg)
    dos = lax.rsqrt(out_deg)

    t1 = dis[:, None] * label
    t2 = dos[:, None] * label
    h1_raw, h2_raw = _conv_plain(
        t1, t2,
        _chunks(col, _EC_CONV), _chunks(row, _EC_CONV, _TRASH),
        _chunks(row, _EC_CONV), _chunks(col, _EC_CONV, _TRASH))
    h1 = dos[:, None] * (h1_raw[:n] - s[:, None] * t1)
    h2 = dis[:, None] * (h2_raw[:n] - s[:, None] * t2)

    t3 = dis[:, None] * h2
    t4 = dos[:, None] * h1
    dis2 = dis * dis
    dos2 = dos * dos
    h3_raw, h4_raw, rea_raw, reb_raw = _conv_scal(
        t3, t4, dis2, dos2,
        _chunks(col, _EC_CONV, 0, _CH_S), _chunks(row, _EC_CONV, _TRASH, _CH_S),
        _chunks(row, _EC_CONV, 0, _CH_S), _chunks(col, _EC_CONV, _TRASH, _CH_S))
    re_a = dos2 * (rea_raw[:n] - s * dis2)
    re_b = dis2 * (reb_raw[:n] - s * dos2)
    h3 = dos[:, None] * (h3_raw[:n] - s[:, None] * t3) - label * re_a[:, None]
    h4 = dis[:, None] * (h4_raw[:n] - s[:, None] * t4) - label * re_b[:, None]

    br = jnp.concatenate([row, col])
    bc = jnp.concatenate([col, row])
    order = jnp.lexsort((bc, br))
    rs = br[order]
    cs = bc[order]
    first = jnp.concatenate(
        [jnp.ones((1,), bool), (rs[1:] != rs[:-1]) | (cs[1:] != cs[:-1])])
    keep = first & (rs != cs)
    trash = jnp.int32(_TRASH)
    rs_k = jnp.where(keep, rs, trash)
    cs_k = jnp.where(keep, cs, trash)

    hh = e
    d0, d1 = _hist1(_chunks(rs_k[:hh], _EC_CONV, _TRASH),
                    _chunks(rs_k[hh:], _EC_CONV, _TRASH))
    deg_u = (d0 + d1)[:n] + 1e-9
    du = lax.rsqrt(deg_u)
    du2 = du * du

    t5 = du[:, None] * label
    p0, p1 = _conv_plain(
        t5, t5,
        _chunks(rs[:hh], _EC_CONV), _chunks(cs_k[:hh], _EC_CONV, _TRASH),
        _chunks(rs[hh:], _EC_CONV), _chunks(cs_k[hh:], _EC_CONV, _TRASH))
    x1 = du[:, None] * (p0 + p1)[:n]

    t6 = du[:, None] * x1
    q0, q1, r3a, r3b = _conv_scal(
        t6, t6, du2, du2,
        _chunks(rs[:hh], _EC_CONV, 0, _CH_S),
        _chunks(cs_k[:hh], _EC_CONV, _TRASH, _CH_S),
        _chunks(rs[hh:], _EC_CONV, 0, _CH_S),
        _chunks(cs_k[hh:], _EC_CONV, _TRASH, _CH_S))
    re3 = du2 * (r3a + r3b)[:n]
    x2 = du[:, None] * (q0 + q1)[:n] - label * re3[:, None]

    return _norm_concat(x2, x1, h3, h2, h1, h4)


# speedup vs baseline: 30.2127x; 1.0032x over previous
import functools

import jax
import jax.numpy as jnp
from jax import lax
from jax.experimental import pallas as pl
from jax.experimental.pallas import tpu as pltpu, tpu_sc as plsc

_N = 100000
_E = 3200000
_CH = 80
_K = 8
_BLK = _CH * _K
_NSC = 16
_RPT = 6272
_NA = _RPT * _NSC
_TRASH = _N
_WB = 392
_EC_CONV = 3276800
_EC_HIST3 = 1638400

_mesh = plsc.VectorSubcoreMesh(core_axis_name="c", subcore_axis_name="s")
_params = pltpu.CompilerParams(use_tc_tiling_on_sc=False)


def _stage_idx(idx_h, buf, sem, sid, nblk, blk):
    off = sid * (nblk * _K) + blk * _K
    return pltpu.async_copy(idx_h.at[pl.ds(off, _K), :], buf, sem)


def _zero_fill2(zb):
    def st(i, c):
        zb[i, :] = jnp.zeros((16,), jnp.float32)
        return c
    lax.fori_loop(0, zb.shape[0], st, 0)


def _zero_fill1(zb):
    def st(i, c):
        zb[pl.ds(i * 16, 16)] = jnp.zeros((16,), jnp.float32)
        return c
    lax.fori_loop(0, zb.shape[0] // 16, st, 0)


def _make_conv(nblk, with_scalar, ch, wb):
    out_type = [jax.ShapeDtypeStruct((_NA, 16), jnp.float32)] * 2
    scratch = [
        pltpu.VMEM_SHARED((_NA, 16), jnp.float32),
        pltpu.VMEM((2, _K, ch), jnp.int32),
        pltpu.VMEM((2, _K, ch), jnp.int32),
        pltpu.VMEM((2, _K, ch, 16), jnp.float32),
        pltpu.VMEM((wb, 16), jnp.float32),
        pltpu.SemaphoreType.DMA,
        pltpu.SemaphoreType.DMA,
        pltpu.SemaphoreType.DMA,
        pltpu.SemaphoreType.DMA,
        pltpu.SemaphoreType.DMA,
        pltpu.SemaphoreType.DMA,
    ]
    if with_scalar:
        out_type += [jax.ShapeDtypeStruct((_NA,), jnp.float32)] * 2
        scratch += [
            pltpu.VMEM_SHARED((_NA,), jnp.float32),
            pltpu.VMEM((2, _K, ch), jnp.float32),
            pltpu.VMEM((wb,), jnp.float32),
        ]
    nhalf = nblk // 2

    @functools.partial(pl.kernel, mesh=_mesh, out_type=out_type,
                       scratch_types=scratch, compiler_params=_params)
    def conv_kernel(*refs):
        if with_scalar:
            (t16a, t16b, t1a, t1b, sA, dA, sB, dB,
             oA, oB, o1A, o1B,
             acc16, sidxv, didxv, rows, zb16,
             sem_i0, sem_i1, sem_g0, sem_g1, sem_s0, sem_s1,
             acc1, scal, zb1) = refs
        else:
            (t16a, t16b, sA, dA, sB, dB,
             oA, oB,
             acc16, sidxv, didxv, rows, zb16,
             sem_i0, sem_i1, sem_g0, sem_g1, sem_s0, sem_s1) = refs
            t1a = t1b = o1A = o1B = acc1 = scal = zb1 = None
        sem_i = (sem_i0, sem_i1)
        sem_g = (sem_g0, sem_g1)
        sem_s = (sem_s0, sem_s1)
        cid = lax.axis_index("c")
        sid = lax.axis_index("s")
        r0 = sid * _RPT

        _zero_fill2(zb16)
        for i in range(_RPT // wb):
            pltpu.sync_copy(zb16, acc16.at[pl.ds(r0 + i * wb, wb), :])
        if with_scalar:
            _zero_fill1(zb1)
            for i in range(_RPT // wb):
                pltpu.sync_copy(zb1, acc1.at[pl.ds(r0 + i * wb, wb)])
        plsc.subcore_barrier()

        def job(t16, t1, sidx_h, didx_h):
            def stage(p, blk):
                off = sid * (nblk * _K) + blk * _K
                h1 = pltpu.async_copy(
                    sidx_h.at[pl.ds(off, _K), :], sidxv.at[p], sem_i[p])
                h2 = pltpu.async_copy(
                    didx_h.at[pl.ds(off, _K), :], didxv.at[p], sem_i[p])
                return h1, h2

            def gathers(p):
                ds = []
                for j in range(_K):
                    ds.append(pltpu.make_async_copy(
                        t16.at[sidxv.at[p, j]], rows.at[p, j], sem_g[p]))
                    if with_scalar:
                        ds.append(pltpu.make_async_copy(
                            t1.at[sidxv.at[p, j]], scal.at[p, j], sem_g[p]))
                return ds

            def scatters(p):
                hs = []
                for j in range(_K):
                    hs.append(pltpu.async_copy(
                        rows.at[p, j], acc16.at[didxv.at[p, j]],
                        sem_s[p], add=True))
                    if with_scalar:
                        hs.append(pltpu.async_copy(
                            scal.at[p, j], acc1.at[didxv.at[p, j]],
                            sem_s[p], add=True))
                return hs

            h1, h2 = stage(0, 0)
            h1.wait()
            h2.wait()
            for d in gathers(0):
                d.start()

            def pair_body(i, carry):
                blk1 = 2 * i + 1
                s1a, s1b = stage(1, blk1)
                for d in gathers(0):
                    d.wait()
                sc0 = scatters(0)
                s1a.wait()
                s1b.wait()
                for d in gathers(1):
                    d.start()
                for h in sc0:
                    h.wait()
                for d in gathers(1):
                    d.wait()
                sc1 = scatters(1)

                @pl.when(i + 1 < nhalf)
                def _():
                    n1, n2 = stage(0, 2 * i + 2)
                    n1.wait()
                    n2.wait()
                    for d in gathers(0):
                        d.start()

                for h in sc1:
                    h.wait()
                return carry

            lax.fori_loop(0, nhalf, pair_body, 0)

        @pl.when(cid == 0)
        def _():
            job(t16a, t1a, sA, dA)

        @pl.when(cid == 1)
        def _():
            job(t16b, t1b, sB, dB)

        plsc.subcore_barrier()

        def writeback(o16, o1):
            for i in range(_RPT // wb):
                pltpu.sync_copy(acc16.at[pl.ds(r0 + i * wb, wb), :], zb16)
                pltpu.sync_copy(zb16, o16.at[pl.ds(r0 + i * wb, wb), :])
            if with_scalar:
                for i in range(_RPT // wb):
                    pltpu.sync_copy(acc1.at[pl.ds(r0 + i * wb, wb)], zb1)
                    pltpu.sync_copy(zb1, o1.at[pl.ds(r0 + i * wb, wb)])

        @pl.when(cid == 0)
        def _():
            writeback(oA, o1A)

        @pl.when(cid == 1)
        def _():
            writeback(oB, o1B)

    return conv_kernel


def _make_hist3(nblk):
    out_type = [jax.ShapeDtypeStruct((_NA,), jnp.float32)] * 6
    scratch = [
        pltpu.VMEM_SHARED((_NA,), jnp.float32),
        pltpu.VMEM_SHARED((_NA,), jnp.float32),
        pltpu.VMEM_SHARED((_NA,), jnp.float32),
        pltpu.VMEM((2, _K, _CH), jnp.int32),
        pltpu.VMEM((2, _K, _CH), jnp.int32),
        pltpu.VMEM((2, _K, _CH), jnp.float32),
        pltpu.VMEM((2, _K, _CH), jnp.float32),
        pltpu.VMEM((_WB,), jnp.float32),
        pltpu.SemaphoreType.DMA,
        pltpu.SemaphoreType.DMA,
        pltpu.SemaphoreType.DMA,
        pltpu.SemaphoreType.DMA,
    ]
    nhalf = nblk // 2

    @functools.partial(pl.kernel, mesh=_mesh, out_type=out_type,
                       scratch_types=scratch, compiler_params=_params)
    def hist_kernel(sA, dA, sB, dB,
                    odA, odB, idA, idB, slA, slB,
                    accA, accB, accC, sidxv, didxv, maskv, selfv, zb1,
                    sem_i0, sem_i1, sem_s0, sem_s1):
        sem_i = (sem_i0, sem_i1)
        sem_s = (sem_s0, sem_s1)
        cid = lax.axis_index("c")
        sid = lax.axis_index("s")
        r0 = sid * _RPT
        _zero_fill1(zb1)
        for i in range(_RPT // _WB):
            pltpu.sync_copy(zb1, accA.at[pl.ds(r0 + i * _WB, _WB)])
            pltpu.sync_copy(zb1, accB.at[pl.ds(r0 + i * _WB, _WB)])
            pltpu.sync_copy(zb1, accC.at[pl.ds(r0 + i * _WB, _WB)])
        plsc.subcore_barrier()

        def job(sidx_h, didx_h):
            def stage(p, blk):
                off = sid * (nblk * _K) + blk * _K
                h1 = pltpu.async_copy(
                    sidx_h.at[pl.ds(off, _K), :], sidxv.at[p], sem_i[p])
                h2 = pltpu.async_copy(
                    didx_h.at[pl.ds(off, _K), :], didxv.at[p], sem_i[p])
                return h1, h2

            def masks(p):
                for j in range(_K):
                    for v in range(_CH // 16):
                        r = sidxv[p, j, pl.ds(v * 16, 16)]
                        c = didxv[p, j, pl.ds(v * 16, 16)]
                        m = jnp.where(r == c, 0.0, 1.0).astype(jnp.float32)
                        maskv[p, j, pl.ds(v * 16, 16)] = m
                        selfv[p, j, pl.ds(v * 16, 16)] = 1.0 - m

            def scatters(p):
                hs = []
                for j in range(_K):
                    hs.append(pltpu.async_copy(
                        maskv.at[p, j], accA.at[sidxv.at[p, j]],
                        sem_s[p], add=True))
                    hs.append(pltpu.async_copy(
                        maskv.at[p, j], accB.at[didxv.at[p, j]],
                        sem_s[p], add=True))
                    hs.append(pltpu.async_copy(
                        selfv.at[p, j], accC.at[sidxv.at[p, j]],
                        sem_s[p], add=True))
                return hs

            h1, h2 = stage(0, 0)
            h1.wait()
            h2.wait()

            def pair_body(i, carry):
                s1a, s1b = stage(1, 2 * i + 1)
                masks(0)
                sc0 = scatters(0)
                s1a.wait()
                s1b.wait()
                masks(1)
                for h in sc0:
                    h.wait()
                sc1 = scatters(1)

                @pl.when(i + 1 < nhalf)
                def _():
                    n1, n2 = stage(0, 2 * i + 2)
                    n1.wait()
                    n2.wait()

                for h in sc1:
                    h.wait()
                return carry

            lax.fori_loop(0, nhalf, pair_body, 0)

        @pl.when(cid == 0)
        def _():
            job(sA, dA)

        @pl.when(cid == 1)
        def _():
            job(sB, dB)

        plsc.subcore_barrier()

        def writeback(od, idg, sl):
            for i in range(_RPT // _WB):
                pltpu.sync_copy(accA.at[pl.ds(r0 + i * _WB, _WB)], zb1)
                pltpu.sync_copy(zb1, od.at[pl.ds(r0 + i * _WB, _WB)])
                pltpu.sync_copy(accB.at[pl.ds(r0 + i * _WB, _WB)], zb1)
                pltpu.sync_copy(zb1, idg.at[pl.ds(r0 + i * _WB, _WB)])
                pltpu.sync_copy(accC.at[pl.ds(r0 + i * _WB, _WB)], zb1)
                pltpu.sync_copy(zb1, sl.at[pl.ds(r0 + i * _WB, _WB)])

        @pl.when(cid == 0)
        def _():
            writeback(odA, idA, slA)

        @pl.when(cid == 1)
        def _():
            writeback(odB, idB, slB)

    return hist_kernel


def _make_hist1(nblk):
    out_type = [jax.ShapeDtypeStruct((_NA,), jnp.float32)] * 2
    scratch = [
        pltpu.VMEM_SHARED((_NA,), jnp.float32),
        pltpu.VMEM((_K, _CH), jnp.int32),
        pltpu.VMEM((_CH,), jnp.float32),
        pltpu.VMEM((_WB,), jnp.float32),
        pltpu.SemaphoreType.DMA,
        pltpu.SemaphoreType.DMA,
    ]

    @functools.partial(pl.kernel, mesh=_mesh, out_type=out_type,
                       scratch_types=scratch, compiler_params=_params)
    def hist1_kernel(iA, iB, oA, oB, acc, idxv, onesv, zb1, sem_i, sem_s):
        cid = lax.axis_index("c")
        sid = lax.axis_index("s")
        r0 = sid * _RPT
        _zero_fill1(zb1)
        for i in range(_RPT // _WB):
            pltpu.sync_copy(zb1, acc.at[pl.ds(r0 + i * _WB, _WB)])
        for v in range(_CH // 16):
            onesv[pl.ds(v * 16, 16)] = jnp.full((16,), 1.0, jnp.float32)
        plsc.subcore_barrier()

        def job(idx_h):
            def blk_body(blk, carry):
                _stage_idx(idx_h, idxv, sem_i, sid, nblk, blk).wait()
                sh = []
                for j in range(_K):
                    sh.append(pltpu.async_copy(
                        onesv, acc.at[idxv.at[j]], sem_s, add=True))
                for h in sh:
                    h.wait()
                return carry
            lax.fori_loop(0, nblk, blk_body, 0)

        @pl.when(cid == 0)
        def _():
            job(iA)

        @pl.when(cid == 1)
        def _():
            job(iB)

        plsc.subcore_barrier()

        def writeback(o):
            for i in range(_RPT // _WB):
                pltpu.sync_copy(acc.at[pl.ds(r0 + i * _WB, _WB)], zb1)
                pltpu.sync_copy(zb1, o.at[pl.ds(r0 + i * _WB, _WB)])

        @pl.when(cid == 0)
        def _():
            writeback(oA)

        @pl.when(cid == 1)
        def _():
            writeback(oB)

    return hist1_kernel


_CH_S = 64
_WB_S = 224
_conv_plain = _make_conv(_EC_CONV // (_NSC * _K * _CH), False, _CH, _WB)
_conv_scal = _make_conv(_EC_CONV // (_NSC * _K * _CH_S), True, _CH_S, _WB_S)
_hist3 = _make_hist3(_EC_HIST3 // (_NSC * _BLK))
_hist1 = _make_hist1(_EC_CONV // (_NSC * _BLK))


def _norm_concat_body(x2, x1, h3, h2, h1, h4, o):
    parts = (x2, x1, h3, h2, h1, h4)
    for i, p in enumerate(parts):
        v = p[...]
        s = jnp.sum(v, axis=1, keepdims=True) + 1e-5
        o[:, i * 16:(i + 1) * 16] = v / s


def _norm_concat(x2, x1, h3, h2, h1, h4):
    n = x2.shape[0]
    rows = 1000
    grid = (n // rows,)
    in_spec = pl.BlockSpec((rows, 16), lambda i: (i, 0))
    out_spec = pl.BlockSpec((rows, 96), lambda i: (i, 0))
    return pl.pallas_call(
        _norm_concat_body,
        grid=grid,
        in_specs=[in_spec] * 6,
        out_specs=out_spec,
        out_shape=jax.ShapeDtypeStruct((n, 96), jnp.float32),
    )(x2, x1, h3, h2, h1, h4)


def _chunks(a, ec_pad, fill=0, ch=_CH):
    pad = ec_pad - a.shape[0]
    if pad:
        a = jnp.concatenate([a, jnp.full((pad,), fill, jnp.int32)])
    return a.reshape(-1, ch)


def kernel(x, label, edge_index, is_direct):
    n = label.shape[0]
    e = edge_index.shape[1]
    row = edge_index[0].astype(jnp.int32)
    col = edge_index[1].astype(jnp.int32)

    h = e // 2
    od0, od1, id0, id1, sl0, sl1 = _hist3(
        _chunks(row[:h], _EC_HIST3, _TRASH), _chunks(col[:h], _EC_HIST3, _TRASH),
        _chunks(row[h:], _EC_HIST3, _TRASH), _chunks(col[h:], _EC_HIST3, _TRASH))
    out_deg = (od0 + od1)[:n] + 1e-9
    in_deg = (id0 + id1)[:n] + 1e-9
    s = (sl0 + sl1)[:n]
    dis = lax.rsqrt(in_deg)
    dos = lax.rsqrt(out_deg)

    t1 = dis[:, None] * label
    t2 = dos[:, None] * label
    h1_raw, h2_raw = _conv_plain(
        t1, t2,
        _chunks(col, _EC_CONV), _chunks(row, _EC_CONV, _TRASH),
        _chunks(row, _EC_CONV), _chunks(col, _EC_CONV, _TRASH))
    h1 = dos[:, None] * (h1_raw[:n] - s[:, None] * t1)
    h2 = dis[:, None] * (h2_raw[:n] - s[:, None] * t2)

    t3 = dis[:, None] * h2
    t4 = dos[:, None] * h1
    dis2 = dis * dis
    dos2 = dos * dos
    h3_raw, h4_raw, rea_raw, reb_raw = _conv_scal(
        t3, t4, dis2, dos2,
        _chunks(col, _EC_CONV, 0, _CH_S), _chunks(row, _EC_CONV, _TRASH, _CH_S),
        _chunks(row, _EC_CONV, 0, _CH_S), _chunks(col, _EC_CONV, _TRASH, _CH_S))
    re_a = dos2 * (rea_raw[:n] - s * dis2)
    re_b = dis2 * (reb_raw[:n] - s * dos2)
    h3 = dos[:, None] * (h3_raw[:n] - s[:, None] * t3) - label * re_a[:, None]
    h4 = dis[:, None] * (h4_raw[:n] - s[:, None] * t4) - label * re_b[:, None]

    br = jnp.concatenate([row, col])
    bc = jnp.concatenate([col, row])
    order = jnp.lexsort((bc, br))
    rs = br[order]
    cs = bc[order]
    first = jnp.concatenate(
        [jnp.ones((1,), bool), (rs[1:] != rs[:-1]) | (cs[1:] != cs[:-1])])
    keep = first & (rs != cs)
    trash = jnp.int32(_TRASH)
    rs_k = jnp.where(keep, rs, trash)
    cs_k = jnp.where(keep, cs, trash)

    hh = e
    d0, d1 = _hist1(_chunks(rs_k[:hh], _EC_CONV, _TRASH),
                    _chunks(rs_k[hh:], _EC_CONV, _TRASH))
    deg_u = (d0 + d1)[:n] + 1e-9
    du = lax.rsqrt(deg_u)
    du2 = du * du

    t5 = du[:, None] * label
    p0, p1 = _conv_plain(
        t5, t5,
        _chunks(rs[:hh], _EC_CONV), _chunks(cs_k[:hh], _EC_CONV, _TRASH),
        _chunks(rs[hh:], _EC_CONV), _chunks(cs_k[hh:], _EC_CONV, _TRASH))
    x1 = du[:, None] * (p0 + p1)[:n]

    t6 = du[:, None] * x1
    q0, q1, r3a, r3b = _conv_scal(
        t6, t6, du2, du2,
        _chunks(rs[:hh], _EC_CONV, 0, _CH_S),
        _chunks(cs_k[:hh], _EC_CONV, _TRASH, _CH_S),
        _chunks(rs[hh:], _EC_CONV, 0, _CH_S),
        _chunks(cs_k[hh:], _EC_CONV, _TRASH, _CH_S))
    re3 = du2 * (r3a + r3b)[:n]
    x2 = du[:, None] * (q0 + q1)[:n] - label * re3[:, None]

    return _norm_concat(x2, x1, h3, h2, h1, h4)
